# baseline (speedup 1.0000x reference)
import functools

import jax
import jax.numpy as jnp
from jax import lax
from jax.experimental import pallas as pl
from jax.experimental.pallas import tpu as pltpu
from jax.experimental.pallas import tpu_sc as plsc

N = 10000
E = 320000
F = 128
H = 2

NC = 2
NS = 16

G = 64
NG = E // G
GC = 8
NCH = NG // GC
NB = 4

ROWS_A = 632
ROWS_B = N - 15 * ROWS_A


def _make_sc_scatter():
    mesh = plsc.VectorSubcoreMesh(core_axis_name="c", subcore_axis_name="s")

    @functools.partial(
        pl.kernel,
        out_type=jax.ShapeDtypeStruct((NC, N, F), jnp.float32),
        mesh=mesh,
        scratch_types=[
            pltpu.VMEM((GC, G), jnp.int32),
            [pltpu.VMEM((G, F), jnp.float32)] * NB,
            pltpu.VMEM_SHARED((N, F), jnp.float32),
            [pltpu.SemaphoreType.DMA] * NB,
            [pltpu.SemaphoreType.DMA] * NB,
        ],
    )
    def sc_scatter(idx_hbm, sv_hbm, zeros_hbm, out_hbm,
                   idx_buf, ubufs, acc, gsems, ssems):
        c = lax.axis_index("c")
        s = lax.axis_index("s")
        col = pl.multiple_of(c * F, F)
        r0 = pl.multiple_of(s * ROWS_A, 8)

        m0 = jnp.where(s < 1, 0, 40 + 39 * (s - 1))
        nch = jnp.where(s < 1, 40, 39)
        g0 = m0 * GC
        ng = nch * GC

        def src_slice(flat):
            e0 = pl.multiple_of((g0 + flat) * G, 8)
            return sv_hbm.at[pl.ds(e0, G), pl.ds(col, F)]

        def gissue(flat, b):
            pltpu.async_copy(src_slice(flat), ubufs[b], gsems[b])

        def gwait(flat, b):
            pltpu.make_async_copy(src_slice(flat), ubufs[b], gsems[b]).wait()

        def sissue(b, j):
            pltpu.async_copy(ubufs[b], acc.at[idx_buf.at[j]], ssems[b],
                             add=True)

        def swait(b):
            pltpu.make_async_copy(ubufs[b], acc.at[idx_buf.at[0]],
                                  ssems[b]).wait()

        gissue(0, 0)
        gissue(1, 1)

        @pl.when(s < NS - 1)
        def _():
            pltpu.sync_copy(zeros_hbm, acc.at[pl.ds(r0, ROWS_A), :])

        @pl.when(s == NS - 1)
        def _():
            pltpu.sync_copy(zeros_hbm.at[pl.ds(0, ROWS_B), :],
                            acc.at[pl.ds(15 * ROWS_A, ROWS_B), :])

        plsc.subcore_barrier()

        def chunk_body(k, carry):
            for j in range(GC):
                b = j % NB
                flat = k * GC + j
                if j == 0:
                    @pl.when(k > 0)
                    def _():
                        swait(2)
                        swait(3)
                    gg = pl.multiple_of((m0 + k) * GC, 8)
                    pltpu.sync_copy(idx_hbm.at[pl.ds(gg, GC), :], idx_buf)
                elif j >= 2:
                    swait((j - 2) % NB)

                @pl.when(flat + 2 < ng)
                def _():
                    gissue(flat + 2, (j + 2) % NB)

                gwait(flat, b)
                sissue(b, j)
            return carry

        lax.fori_loop(0, nch, chunk_body, 0)
        swait(2)
        swait(3)

        plsc.subcore_barrier()

        @pl.when(s < NS - 1)
        def _():
            pltpu.sync_copy(acc.at[pl.ds(r0, ROWS_A), :],
                            out_hbm.at[c, pl.ds(r0, ROWS_A), :])

        @pl.when(s == NS - 1)
        def _():
            pltpu.sync_copy(acc.at[pl.ds(15 * ROWS_A, ROWS_B), :],
                            out_hbm.at[c, pl.ds(15 * ROWS_A, ROWS_B), :])

    return sc_scatter


_sc_scatter = _make_sc_scatter()


def _mm_body(xp_ref, x_ref, w_ref, b_ref, o_ref):
    o_ref[...] = (
        x_ref[...]
        + b_ref[...]
        + jnp.dot(xp_ref[0], w_ref[0], preferred_element_type=jnp.float32)
        + jnp.dot(xp_ref[1], w_ref[1], preferred_element_type=jnp.float32)
    )


def _tc_dense(xp, x_i, W2, b2):
    BN = 2000
    grid = (N // BN,)
    return pl.pallas_call(
        _mm_body,
        grid=grid,
        in_specs=[
            pl.BlockSpec((H, BN, F), lambda i: (0, i, 0)),
            pl.BlockSpec((BN, F), lambda i: (i, 0)),
            pl.BlockSpec((H, F, F), lambda i: (0, 0, 0)),
            pl.BlockSpec((1, F), lambda i: (0, 0)),
        ],
        out_specs=pl.BlockSpec((BN, F), lambda i: (i, 0)),
        out_shape=jax.ShapeDtypeStruct((N, F), jnp.float32),
    )(xp, x_i, W2, b2)


def kernel(nbrs, x_i, scaled_v, W, b):
    idx2d = nbrs[:, 0].astype(jnp.int32).reshape(NG, G)
    zeros = jnp.zeros((ROWS_A, F), jnp.float32)
    xp = _sc_scatter(idx2d, scaled_v, zeros)
    return _tc_dense(xp, x_i, W.reshape(H, F, F), b.reshape(1, F))


# speedup vs baseline: 1.0333x; 1.0333x over previous
import functools

import jax
import jax.numpy as jnp
from jax import lax
from jax.experimental import pallas as pl
from jax.experimental.pallas import tpu as pltpu
from jax.experimental.pallas import tpu_sc as plsc

N = 10000
E = 320000
F = 128
H = 2

NC = 2
NS = 16

G = 80
NG = E // G
GC = 8
NCH = NG // GC
NB = 4

ROWS_A = 632
ROWS_B = N - 15 * ROWS_A


def _make_sc_scatter():
    mesh = plsc.VectorSubcoreMesh(core_axis_name="c", subcore_axis_name="s")

    @functools.partial(
        pl.kernel,
        out_type=jax.ShapeDtypeStruct((NC, N, F), jnp.float32),
        mesh=mesh,
        scratch_types=[
            pltpu.VMEM((GC, G), jnp.int32),
            [pltpu.VMEM((G, F), jnp.float32)] * NB,
            pltpu.VMEM_SHARED((N, F), jnp.float32),
            [pltpu.SemaphoreType.DMA] * NB,
            [pltpu.SemaphoreType.DMA] * NB,
        ],
    )
    def sc_scatter(idx_hbm, sv_hbm, zeros_hbm, out_hbm,
                   idx_buf, ubufs, acc, gsems, ssems):
        c = lax.axis_index("c")
        s = lax.axis_index("s")
        col = pl.multiple_of(c * F, F)
        r0 = pl.multiple_of(s * ROWS_A, 8)

        m0 = jnp.where(s < 4, 32 * s, 128 + 31 * (s - 4))
        nch = jnp.where(s < 4, 32, 31)
        g0 = m0 * GC
        ng = nch * GC

        def src_slice(flat):
            e0 = pl.multiple_of((g0 + flat) * G, 8)
            return sv_hbm.at[pl.ds(e0, G), pl.ds(col, F)]

        def gissue(flat, b):
            pltpu.async_copy(src_slice(flat), ubufs[b], gsems[b])

        def gwait(flat, b):
            pltpu.make_async_copy(src_slice(flat), ubufs[b], gsems[b]).wait()

        def sissue(b, j):
            pltpu.async_copy(ubufs[b], acc.at[idx_buf.at[j]], ssems[b],
                             add=True)

        def swait(b):
            pltpu.make_async_copy(ubufs[b], acc.at[idx_buf.at[0]],
                                  ssems[b]).wait()

        gissue(0, 0)
        gissue(1, 1)

        @pl.when(s < NS - 1)
        def _():
            pltpu.sync_copy(zeros_hbm, acc.at[pl.ds(r0, ROWS_A), :])

        @pl.when(s == NS - 1)
        def _():
            pltpu.sync_copy(zeros_hbm.at[pl.ds(0, ROWS_B), :],
                            acc.at[pl.ds(15 * ROWS_A, ROWS_B), :])

        plsc.subcore_barrier()

        def chunk_body(k, carry):
            for j in range(GC):
                b = j % NB
                flat = k * GC + j
                if j == 0:
                    @pl.when(k > 0)
                    def _():
                        swait(2)
                        swait(3)
                    gg = pl.multiple_of((m0 + k) * GC, 8)
                    pltpu.sync_copy(idx_hbm.at[pl.ds(gg, GC), :], idx_buf)
                elif j >= 2:
                    swait((j - 2) % NB)

                @pl.when(flat + 2 < ng)
                def _():
                    gissue(flat + 2, (j + 2) % NB)

                gwait(flat, b)
                sissue(b, j)
            return carry

        lax.fori_loop(0, nch, chunk_body, 0)
        swait(2)
        swait(3)

        plsc.subcore_barrier()

        @pl.when(s < NS - 1)
        def _():
            pltpu.sync_copy(acc.at[pl.ds(r0, ROWS_A), :],
                            out_hbm.at[c, pl.ds(r0, ROWS_A), :])

        @pl.when(s == NS - 1)
        def _():
            pltpu.sync_copy(acc.at[pl.ds(15 * ROWS_A, ROWS_B), :],
                            out_hbm.at[c, pl.ds(15 * ROWS_A, ROWS_B), :])

    return sc_scatter


_sc_scatter = _make_sc_scatter()


def _mm_body(xp_ref, x_ref, w_ref, b_ref, o_ref):
    o_ref[...] = (
        x_ref[...]
        + b_ref[...]
        + jnp.dot(xp_ref[0], w_ref[0], preferred_element_type=jnp.float32)
        + jnp.dot(xp_ref[1], w_ref[1], preferred_element_type=jnp.float32)
    )


def _tc_dense(xp, x_i, W2, b2):
    BN = 2000
    grid = (N // BN,)
    return pl.pallas_call(
        _mm_body,
        grid=grid,
        in_specs=[
            pl.BlockSpec((H, BN, F), lambda i: (0, i, 0)),
            pl.BlockSpec((BN, F), lambda i: (i, 0)),
            pl.BlockSpec((H, F, F), lambda i: (0, 0, 0)),
            pl.BlockSpec((1, F), lambda i: (0, 0)),
        ],
        out_specs=pl.BlockSpec((BN, F), lambda i: (i, 0)),
        out_shape=jax.ShapeDtypeStruct((N, F), jnp.float32),
    )(xp, x_i, W2, b2)


def kernel(nbrs, x_i, scaled_v, W, b):
    idx2d = nbrs[:, 0].astype(jnp.int32).reshape(NG, G)
    zeros = jnp.zeros((ROWS_A, F), jnp.float32)
    xp = _sc_scatter(idx2d, scaled_v, zeros)
    return _tc_dense(xp, x_i, W.reshape(H, F, F), b.reshape(1, F))
